# baseline (speedup 1.0000x reference)
import functools

import jax
import jax.numpy as jnp
from jax import lax
from jax.experimental import pallas as pl
from jax.experimental.pallas import tpu as pltpu
from jax.experimental.pallas import tpu_sc as plsc

_NC = 2
_NS = 16
_NW = _NC * _NS
_NBUF = 4


@functools.partial(jax.jit, static_argnums=(2, 3, 4))
def _sc_gather(table, idx, b_per_w, nchunks, chunk):
    vocab, d = table.shape
    b_total = _NW * b_per_w
    nrounds = nchunks // _NBUF
    mesh = plsc.VectorSubcoreMesh(core_axis_name="c", subcore_axis_name="s")

    @functools.partial(
        pl.kernel,
        mesh=mesh,
        out_type=jax.ShapeDtypeStruct((b_total, d), jnp.float32),
        scratch_types=[
            pltpu.VMEM((nchunks, chunk), jnp.int32),
            [pltpu.VMEM((chunk, d), jnp.float32) for _ in range(_NBUF)],
            [pltpu.SemaphoreType.DMA for _ in range(_NBUF)],
            [pltpu.SemaphoreType.DMA for _ in range(_NBUF)],
        ],
    )
    def k(table_hbm, idx_hbm, out_hbm, idx_v, bufs, gsems, wsems):
        wid = lax.axis_index("s") * _NC + lax.axis_index("c")
        base = wid * b_per_w
        pltpu.sync_copy(idx_hbm.at[wid], idx_v)
        for b in range(_NBUF):
            pltpu.async_copy(table_hbm.at[idx_v.at[b]], bufs[b], gsems[b])

        @pl.loop(0, nrounds)
        def _(t):
            c = t * _NBUF
            for b in range(_NBUF):
                pltpu.make_async_copy(
                    table_hbm.at[idx_v.at[c + b]], bufs[b], gsems[b]).wait()
                pltpu.async_copy(
                    bufs[b],
                    out_hbm.at[pl.ds(base + (c + b) * chunk, chunk)],
                    wsems[b])

            @pl.when(t + 1 < nrounds)
            def _():
                for b in range(_NBUF):
                    pltpu.make_async_copy(
                        bufs[b], out_hbm.at[pl.ds(base, chunk)],
                        wsems[b]).wait()
                    pltpu.async_copy(
                        table_hbm.at[idx_v.at[c + _NBUF + b]],
                        bufs[b], gsems[b])

        for b in range(_NBUF):
            pltpu.make_async_copy(
                bufs[b], out_hbm.at[pl.ds(base, chunk)], wsems[b]).wait()

    return k(table, idx)


def kernel(timesteps, table):
    b, l, _ = timesteps.shape
    d = table.shape[1]
    n = b * l
    b_per_w = n // _NW
    chunk = 40
    nchunks = b_per_w // chunk
    idx = timesteps.astype(jnp.int32).reshape(_NW, nchunks, chunk)
    out = _sc_gather(table, idx, b_per_w, nchunks, chunk)
    return out.reshape(b, l, d)
